# baseline (speedup 1.0000x reference)
import functools
import math

import jax
import jax.numpy as jnp
from jax import lax
from jax.experimental import pallas as pl
from jax.experimental.pallas import tpu as pltpu
from jax.experimental.pallas import tpu_sc as plsc

VOCAB = 250027
D = 1024
B = 32
S = 1024
L = 16
NV = D // L
NC = 2
NS = 16
NW = NC * NS
CHUNKS = S // NW
RG = 8
SCALE = math.sqrt(float(D))
EPS = 1e-5


def _lane_sum(x):
    dnums = lax.GatherDimensionNumbers(
        offset_dims=(), collapsed_slice_dims=(0,), start_index_map=(0,))
    for sh in (8, 4, 2, 1):
        idx = (lax.iota(jnp.int32, L) + sh) % L
        x = x + lax.gather(x, idx[:, None], dnums, slice_sizes=(1,),
                           mode=lax.GatherScatterMode.PROMISE_IN_BOUNDS)
    return x


def _rsqrt16(a):
    seed = lax.bitcast_convert_type(
        jnp.int32(0x5F3759DF) - (lax.bitcast_convert_type(a, jnp.int32) >> 1),
        jnp.float32)
    y = seed
    for _ in range(3):
        y = y * (1.5 - 0.5 * a * y * y)
    return y


def _sc_body(tok_hbm, weight_hbm, pos_hbm, gam_hbm, bet_hbm, out_hbm,
             tok_v, pos_v, gam_v, bet_v, prow, buf0, buf1,
             gs0, gs1, ss0, ss1):
    wid = lax.axis_index("s") * NC + lax.axis_index("c")
    row0 = wid * CHUNKS * B

    pltpu.sync_copy(tok_hbm.at[pl.ds(wid * CHUNKS, CHUNKS)], tok_v)
    pltpu.sync_copy(pos_hbm.at[pl.ds(wid * CHUNKS, CHUNKS)], pos_v)
    pltpu.sync_copy(gam_hbm, gam_v)
    pltpu.sync_copy(bet_hbm, bet_v)

    bufs = (buf0, buf1)
    gsems = (gs0, gs1)
    ssems = (ss0, ss1)

    pltpu.async_copy(weight_hbm.at[tok_v.at[0]], buf0, gs0)
    pltpu.async_copy(weight_hbm.at[tok_v.at[1]], buf1, gs1)

    zeros = jnp.zeros((L,), jnp.float32)

    def ln_chunk(c, buf):
        def pcopy(j, _):
            sl = pl.ds(j * L, L)
            prow[sl] = pos_v[c, sl]
            return 0

        lax.fori_loop(0, NV, pcopy, 0)

        for g0 in range(0, B, RG):
            def p1(j, accs):
                sl = pl.ds(j * L, L)
                pvec = prow[sl]
                out = []
                for t in range(RG):
                    x = buf[g0 + t, sl] * SCALE + pvec
                    buf[g0 + t, sl] = x
                    out.append(accs[2 * t] + x)
                    out.append(accs[2 * t + 1] + x * x)
                return tuple(out)

            accs = lax.fori_loop(0, NV, p1, (zeros,) * (2 * RG))

            stats = []
            for t in range(RG):
                mean = _lane_sum(accs[2 * t]) * (1.0 / D)
                var = _lane_sum(accs[2 * t + 1]) * (1.0 / D) - mean * mean
                stats.append((mean, _rsqrt16(var + EPS)))

            def p2(j, _):
                sl = pl.ds(j * L, L)
                gam = gam_v[sl]
                bet = bet_v[sl]
                for t in range(RG):
                    mean, rstd = stats[t]
                    y = (buf[g0 + t, sl] - mean) * rstd
                    buf[g0 + t, sl] = y * gam + bet
                return 0

            lax.fori_loop(0, NV, p2, 0)

    def pair_body(i, _):
        for b in range(2):
            c = 2 * i + b
            buf, gs, ss = bufs[b], gsems[b], ssems[b]
            pltpu.make_async_copy(weight_hbm.at[tok_v.at[c]], buf, gs).wait()
            ln_chunk(c, buf)
            dst = out_hbm.at[pl.ds(row0 + c * B, B)]
            pltpu.async_copy(buf, dst, ss)

            @pl.when(c + 2 < CHUNKS)
            def _():
                pltpu.make_async_copy(buf, dst, ss).wait()
                pltpu.async_copy(weight_hbm.at[tok_v.at[c + 2]], buf, gs)

        return 0

    lax.fori_loop(0, CHUNKS // 2, pair_body, 0)

    for b in range(2):
        c = CHUNKS - 2 + b
        pltpu.make_async_copy(bufs[b], out_hbm.at[pl.ds(row0 + c * B, B)],
                              ssems[b]).wait()


@jax.jit
def _shard_head_tail(tokens, weight, pos_weight, ln_gamma, ln_beta):
    tokens_t = jnp.transpose(tokens)
    mesh = plsc.VectorSubcoreMesh(core_axis_name="c", subcore_axis_name="s")
    out = pl.kernel(
        _sc_body,
        mesh=mesh,
        out_type=jax.ShapeDtypeStruct((S * B, D), jnp.float32),
        scratch_types=[
            pltpu.VMEM((CHUNKS, B), jnp.int32),
            pltpu.VMEM((CHUNKS, D), jnp.float32),
            pltpu.VMEM((D,), jnp.float32),
            pltpu.VMEM((D,), jnp.float32),
            pltpu.VMEM((D,), jnp.float32),
            pltpu.VMEM((B, D), jnp.float32),
            pltpu.VMEM((B, D), jnp.float32),
            pltpu.SemaphoreType.DMA,
            pltpu.SemaphoreType.DMA,
            pltpu.SemaphoreType.DMA,
            pltpu.SemaphoreType.DMA,
        ],
    )(tokens_t, weight, pos_weight, ln_gamma, ln_beta)
    return out.reshape(S, B, D)


def kernel(tokens, weight, pos_weight, ln_gamma, ln_beta):
    return _shard_head_tail(tokens, weight, pos_weight, ln_gamma, ln_beta)


# speedup vs baseline: 1.6447x; 1.6447x over previous
import functools
import math

import jax
import jax.numpy as jnp
from jax import lax
from jax.experimental import pallas as pl
from jax.experimental.pallas import tpu as pltpu
from jax.experimental.pallas import tpu_sc as plsc

VOCAB = 250027
D = 1024
B = 32
S = 1024
L = 16
NV = D // L
NC = 2
NS = 16
NW = NC * NS
CHUNKS = S // NW
HB = B // 2
SCALE = math.sqrt(float(D))
EPS = 1e-5
SB = 64
RND = 0x8000
HIMASK = -65536


def _sc_gather_pack_body(tok_hbm, weight_hbm, out_hbm,
                         tok_v, buf0, buf1, pk0, pk1, gs0, gs1, ss0, ss1):
    wid = lax.axis_index("s") * NC + lax.axis_index("c")
    row0 = wid * CHUNKS * HB

    pltpu.sync_copy(tok_hbm.at[pl.ds(wid * CHUNKS, CHUNKS)], tok_v)

    bufs = (buf0, buf1)
    pks = (pk0, pk1)
    gsems = (gs0, gs1)
    ssems = (ss0, ss1)

    pltpu.async_copy(weight_hbm.at[tok_v.at[0]], buf0, gs0)
    pltpu.async_copy(weight_hbm.at[tok_v.at[1]], buf1, gs1)

    def pack_chunk(buf, pk):
        def jbody(j, _):
            sl = pl.ds(j * L, L)
            for r in range(HB):
                ua = lax.bitcast_convert_type(buf[r, sl], jnp.int32)
                ub = lax.bitcast_convert_type(buf[r + HB, sl], jnp.int32)
                lo = lax.shift_right_logical(ua + RND, 16)
                hi = (ub + RND) & HIMASK
                pk[r, sl] = hi | lo
            return 0

        lax.fori_loop(0, NV, jbody, 0)

    def pair_body(i, _):
        for b in range(2):
            c = 2 * i + b
            buf, pk, gs, ss = bufs[b], pks[b], gsems[b], ssems[b]
            pltpu.make_async_copy(weight_hbm.at[tok_v.at[c]], buf, gs).wait()
            dst = out_hbm.at[pl.ds(row0 + c * HB, HB)]

            @pl.when(c >= 2)
            def _():
                pltpu.make_async_copy(
                    pk, out_hbm.at[pl.ds(row0 + (c - 2) * HB, HB)], ss).wait()

            pack_chunk(buf, pk)
            pltpu.async_copy(pk, dst, ss)

            @pl.when(c + 2 < CHUNKS)
            def _():
                pltpu.async_copy(weight_hbm.at[tok_v.at[c + 2]], buf, gs)

        return 0

    lax.fori_loop(0, CHUNKS // 2, pair_body, 0)

    for b in range(2):
        c = CHUNKS - 2 + b
        pltpu.make_async_copy(pks[b], out_hbm.at[pl.ds(row0 + c * HB, HB)],
                              ssems[b]).wait()


def _sc_gather_pack(tokens_t, weight):
    mesh = plsc.VectorSubcoreMesh(core_axis_name="c", subcore_axis_name="s")
    return pl.kernel(
        _sc_gather_pack_body,
        mesh=mesh,
        out_type=jax.ShapeDtypeStruct((S * HB, D), jnp.int32),
        scratch_types=[
            pltpu.VMEM((CHUNKS, B), jnp.int32),
            pltpu.VMEM((B, D), jnp.float32),
            pltpu.VMEM((B, D), jnp.float32),
            pltpu.VMEM((HB, D), jnp.int32),
            pltpu.VMEM((HB, D), jnp.int32),
            pltpu.SemaphoreType.DMA,
            pltpu.SemaphoreType.DMA,
            pltpu.SemaphoreType.DMA,
            pltpu.SemaphoreType.DMA,
        ],
    )(tokens_t, weight)


def _tc_ln_body(pos_ref, gam_ref, bet_ref, pk_ref, o_ref):
    w = pk_ref[...].reshape(SB, HB, D)
    xlo = lax.bitcast_convert_type(w << 16, jnp.float32)
    xhi = lax.bitcast_convert_type(w & HIMASK, jnp.float32)
    x = jnp.concatenate([xlo, xhi], axis=1)
    x = x * SCALE + pos_ref[...][:, None, :]
    mean = jnp.mean(x, axis=-1, keepdims=True)
    xc = x - mean
    var = jnp.mean(xc * xc, axis=-1, keepdims=True)
    y = xc * lax.rsqrt(var + EPS) * gam_ref[...][None, :, :] + bet_ref[...]
    o_ref[...] = y.reshape(SB * B, D)


def _tc_ln(pk, pos_weight, gamma2d, beta2d):
    return pl.pallas_call(
        _tc_ln_body,
        grid=(S // SB,),
        in_specs=[
            pl.BlockSpec((SB, D), lambda i: (i, 0)),
            pl.BlockSpec((1, D), lambda i: (0, 0)),
            pl.BlockSpec((1, D), lambda i: (0, 0)),
            pl.BlockSpec((SB * HB, D), lambda i: (i, 0)),
        ],
        out_specs=pl.BlockSpec((SB * B, D), lambda i: (i, 0)),
        out_shape=jax.ShapeDtypeStruct((S * B, D), jnp.float32),
        compiler_params=pltpu.CompilerParams(
            dimension_semantics=("arbitrary",)),
    )(pos_weight, gamma2d, beta2d, pk)


@jax.jit
def _shard_head_tail(tokens, weight, pos_weight, ln_gamma, ln_beta):
    tokens_t = jnp.transpose(tokens)
    pk = _sc_gather_pack(tokens_t, weight)
    out = _tc_ln(pk, pos_weight,
                 ln_gamma.reshape(1, D), ln_beta.reshape(1, D))
    return out.reshape(S, B, D)


def kernel(tokens, weight, pos_weight, ln_gamma, ln_beta):
    return _shard_head_tail(tokens, weight, pos_weight, ln_gamma, ln_beta)
